# baseline (speedup 1.0000x reference)
import functools

import jax
import jax.numpy as jnp
from jax import lax
from jax.experimental import pallas as pl
from jax.experimental.pallas import tpu as pltpu
from jax.experimental.pallas import tpu_sc as plsc

CBE = 0.1
L = 16
NC = 2
NS = 16
NW = NC * NS


def _table_body(counts_ref, out_ref):
    out_ref[...] = CBE * lax.rsqrt(counts_ref[...] + 1.0)


def _make_sc_kernel(n, b0, b1, per_w):
    mesh = plsc.VectorSubcoreMesh(core_axis_name="c", subcore_axis_name="s")

    ch = min(per_w, 8192)
    n_ch = per_w // ch
    assert per_w % ch == 0 and ch % L == 0

    @functools.partial(
        pl.kernel,
        mesh=mesh,
        out_type=jax.ShapeDtypeStruct((n,), jnp.float32),
        compiler_params=pltpu.CompilerParams(
            needs_layout_passes=False, use_tc_tiling_on_sc=False),
        scratch_types=[
            pltpu.VMEM((b0, b1), jnp.float32),
            pltpu.VMEM((ch, 2), jnp.float32),
            pltpu.VMEM((per_w,), jnp.float32),
            pltpu.VMEM((4 * L,), jnp.float32),
        ],
    )
    def sc_kernel(table_hbm, obs_hbm, params_hbm, out_hbm,
                  table_v, obs_v, out_v, params_v):
        wid = lax.axis_index("s") * NC + lax.axis_index("c")
        base = wid * per_w
        pltpu.sync_copy(table_hbm, table_v)
        pltpu.sync_copy(params_hbm, params_v)

        scale_x = params_v[pl.ds(0 * L, L)]
        scale_y = params_v[pl.ds(1 * L, L)]
        hi_x = params_v[pl.ds(2 * L, L)]
        hi_y = params_v[pl.ds(3 * L, L)]
        iota = lax.iota(jnp.int32, L)
        zero = jnp.zeros((L,), jnp.int32)
        one = jnp.ones((L,), jnp.int32)

        for c in range(n_ch):
            pltpu.sync_copy(obs_hbm.at[pl.ds(base + c * ch, ch)], obs_v)

            @plsc.parallel_loop(0, ch // L, unroll=16)
            def body(j):
                rows = j * L + iota
                x = plsc.load_gather(obs_v, [rows, zero])
                y = plsc.load_gather(obs_v, [rows, one])
                sx = x * scale_x
                sy = y * scale_y
                sx = jnp.where(sx >= hi_x, sx - 1.0, sx)
                sy = jnp.where(sy >= hi_y, sy - 1.0, sy)
                bx = sx.astype(jnp.int32)
                by = sy.astype(jnp.int32)
                out_v[pl.ds(c * ch + j * L, L)] = plsc.load_gather(
                    table_v, [bx, by])

        pltpu.sync_copy(out_v, out_hbm.at[pl.ds(base, per_w)])

    return sc_kernel


def kernel(ob_no, counts, obs_low, obs_high):
    n, obs_dim = ob_no.shape
    b0, b1 = counts.shape
    assert obs_dim == 2
    assert n % (NW * L) == 0
    per_w = n // NW

    bonus = pl.pallas_call(
        _table_body,
        out_shape=jax.ShapeDtypeStruct((b0, b1), jnp.float32),
    )(counts)

    scale = obs_high - obs_low
    params = jnp.concatenate([
        jnp.full((L,), scale[0], jnp.float32),
        jnp.full((L,), scale[1], jnp.float32),
        jnp.full((L,), obs_high[0], jnp.float32),
        jnp.full((L,), obs_high[1], jnp.float32),
    ])

    sc = _make_sc_kernel(n, b0, b1, per_w)
    return sc(bonus, ob_no, params)


# speedup vs baseline: 41.5264x; 41.5264x over previous
import functools

import jax
import jax.numpy as jnp
from jax import lax
from jax.experimental import pallas as pl
from jax.experimental.pallas import tpu as pltpu
from jax.experimental.pallas import tpu_sc as plsc

CBE = 0.1
L = 16
NC = 2
NS = 16
NW = NC * NS
BLK = 128


def _table_body(counts_ref, out_ref):
    out_ref[...] = CBE * lax.rsqrt(counts_ref[...] + 1.0)


def _make_sc_kernel(n, b0, b1, per_w):
    mesh = plsc.VectorSubcoreMesh(core_axis_name="c", subcore_axis_name="s")
    blocks_per_w = per_w // BLK

    @functools.partial(
        pl.kernel,
        mesh=mesh,
        out_type=jax.ShapeDtypeStruct((n,), jnp.float32),
        compiler_params=pltpu.CompilerParams(
            needs_layout_passes=False, use_tc_tiling_on_sc=False),
        scratch_types=[
            pltpu.VMEM((b0, b1), jnp.float32),
            pltpu.VMEM((blocks_per_w, 2, BLK), jnp.float32),
            pltpu.VMEM((per_w,), jnp.float32),
            pltpu.VMEM((4 * L,), jnp.float32),
        ],
    )
    def sc_kernel(table_hbm, obs_hbm, params_hbm, out_hbm,
                  table_v, obs_v, out_v, params_v):
        wid = lax.axis_index("s") * NC + lax.axis_index("c")
        base_b = wid * blocks_per_w
        pltpu.sync_copy(table_hbm, table_v)
        pltpu.sync_copy(params_hbm, params_v)
        pltpu.sync_copy(obs_hbm.at[pl.ds(base_b, blocks_per_w)], obs_v)

        scale_x = params_v[pl.ds(0 * L, L)]
        scale_y = params_v[pl.ds(1 * L, L)]
        hi_x = params_v[pl.ds(2 * L, L)]
        hi_y = params_v[pl.ds(3 * L, L)]

        @plsc.parallel_loop(0, blocks_per_w, unroll=4)
        def body(b):
            for g in range(BLK // L):
                x = obs_v[b, 0, pl.ds(g * L, L)]
                y = obs_v[b, 1, pl.ds(g * L, L)]
                sx = x * scale_x
                sy = y * scale_y
                sx = jnp.where(sx >= hi_x, sx - 1.0, sx)
                sy = jnp.where(sy >= hi_y, sy - 1.0, sy)
                bx = sx.astype(jnp.int32)
                by = sy.astype(jnp.int32)
                out_v[pl.ds(b * BLK + g * L, L)] = plsc.load_gather(
                    table_v, [bx, by])

        pltpu.sync_copy(out_v, out_hbm.at[pl.ds(wid * per_w, per_w)])

    return sc_kernel


def kernel(ob_no, counts, obs_low, obs_high):
    n, obs_dim = ob_no.shape
    b0, b1 = counts.shape
    assert obs_dim == 2
    assert n % (NW * BLK) == 0
    per_w = n // NW

    bonus = pl.pallas_call(
        _table_body,
        out_shape=jax.ShapeDtypeStruct((b0, b1), jnp.float32),
    )(counts)

    scale = obs_high - obs_low
    params = jnp.concatenate([
        jnp.full((L,), scale[0], jnp.float32),
        jnp.full((L,), scale[1], jnp.float32),
        jnp.full((L,), obs_high[0], jnp.float32),
        jnp.full((L,), obs_high[1], jnp.float32),
    ])

    obs_blocked = ob_no.reshape(n // BLK, BLK, 2).transpose(0, 2, 1)

    sc = _make_sc_kernel(n, b0, b1, per_w)
    return sc(bonus, obs_blocked, params)
